# baseline (speedup 1.0000x reference)
import functools

import jax
import jax.numpy as jnp
from jax import lax
from jax.experimental import pallas as pl
from jax.experimental.pallas import tpu as pltpu
from jax.experimental.pallas import tpu_sc as plsc

_NC = 2
_NS = 16
_NW = _NC * _NS
_B = 128
_NCHUNK = 2
_RBLK = 512


def _make_sc_segsum(n, d, n_pad, nb):
  rows_per_sub = n_pad // _NS
  hc = nb // _NCHUNK
  mesh = plsc.VectorSubcoreMesh(core_axis_name="c", subcore_axis_name="s")

  @functools.partial(
      pl.kernel,
      out_type=(
          jax.ShapeDtypeStruct((_NC, n_pad, d), jnp.float32),
          jax.ShapeDtypeStruct((_NC, n_pad), jnp.float32),
      ),
      mesh=mesh,
      scratch_types=[
          pltpu.VMEM((hc, _B), jnp.int32),
          pltpu.VMEM((hc, _B), jnp.int32),
          pltpu.VMEM((_B, d), jnp.float32),
          pltpu.VMEM((_B, d), jnp.float32),
          pltpu.VMEM((_B,), jnp.float32),
          pltpu.VMEM_SHARED((n_pad, d), jnp.float32),
          pltpu.VMEM_SHARED((n_pad,), jnp.float32),
          pltpu.SemaphoreType.DMA,
          pltpu.SemaphoreType.DMA,
          pltpu.SemaphoreType.DMA,
      ],
  )
  def sc_segsum(x_h, src_h, dst_h, zr_h, zd_h, part_h, degp_h,
                src_v, dst_v, rows_v0, rows_v1, ones_v, acc_sh, deg_sh,
                sem0, sem1, sem_deg):
    c = lax.axis_index("c")
    s = lax.axis_index("s")
    w = c * _NS + s

    pltpu.sync_copy(zr_h.at[pl.ds(s * rows_per_sub, rows_per_sub)],
                    acc_sh.at[pl.ds(s * rows_per_sub, rows_per_sub)])

    @pl.when(s == 0)
    def _zero_deg():
      pltpu.sync_copy(zd_h, deg_sh)

    for k in range(_B // 16):
      ones_v[pl.ds(16 * k, 16)] = jnp.ones((16,), jnp.float32)

    plsc.subcore_barrier()

    rows_bufs = (rows_v0, rows_v1)
    sems = (sem0, sem1)

    for h in range(_NCHUNK):
      pltpu.sync_copy(src_h.at[w, pl.ds(h * hc, hc)], src_v)
      pltpu.sync_copy(dst_h.at[w, pl.ds(h * hc, hc)], dst_v)

      pltpu.async_copy(x_h.at[src_v.at[0]], rows_v0, sem0)
      pltpu.async_copy(x_h.at[src_v.at[1]], rows_v1, sem1)

      def body(t, carry):
        for slot in range(2):
          i = 2 * t + slot
          rv, sm = rows_bufs[slot], sems[slot]
          pltpu.make_async_copy(x_h.at[src_v.at[i]], rv, sm).wait()
          pltpu.sync_copy(rv, acc_sh.at[dst_v.at[i]], add=True)
          pltpu.async_copy(ones_v, deg_sh.at[dst_v.at[i]], sem_deg, add=True)

          @pl.when(t < hc // 2 - 1)
          def _next():
            pltpu.async_copy(x_h.at[src_v.at[i + 2]], rv, sm)

        return carry

      lax.fori_loop(0, hc // 2, body, 0)

      def drain(i, carry):
        pltpu.make_async_copy(ones_v, deg_sh.at[dst_v.at[0]], sem_deg).wait()
        return carry

      lax.fori_loop(0, hc, drain, 0)

    plsc.subcore_barrier()

    pltpu.sync_copy(acc_sh.at[pl.ds(s * rows_per_sub, rows_per_sub)],
                    part_h.at[c, pl.ds(s * rows_per_sub, rows_per_sub)])

    @pl.when(s == 0)
    def _write_deg():
      pltpu.sync_copy(deg_sh, degp_h.at[c])

  return sc_segsum


def _tc_finish(p0_ref, p1_ref, d0_ref, d1_ref, w_ref, b_ref, o_ref):
  ssum = p0_ref[...] + p1_ref[...]
  deg = d0_ref[...] + d1_ref[...]
  inv = 1.0 / jnp.maximum(deg, 1.0)
  sn = ssum * inv
  h = jnp.dot(sn, w_ref[...], preferred_element_type=jnp.float32)
  h = h + b_ref[...]
  o_ref[...] = jnp.where(h >= 0.0, h, 0.01 * h)


def kernel(x, edge_index, W, b):
  n, d = x.shape
  e = edge_index.shape[1]
  ept = -(-e // _NW)
  nb = -(-ept // _B)
  nb += -nb % (2 * _NCHUNK)
  e_pad = _NW * nb * _B
  n_pad = (n // _RBLK + 1) * _RBLK

  src = edge_index[0]
  dst = edge_index[1]
  pad = e_pad - e
  src_r = jnp.pad(src, (0, pad)).reshape(_NW, nb, _B)
  dst_r = jnp.pad(dst, (0, pad), constant_values=n).reshape(_NW, nb, _B)
  zrows = jnp.zeros((n_pad, d), jnp.float32)
  zdeg = jnp.zeros((n_pad,), jnp.float32)

  part, degp = _make_sc_segsum(n, d, n_pad, nb)(x, src_r, dst_r, zrows, zdeg)

  grid = n_pad // _RBLK
  d0 = degp[0].reshape(n_pad, 1)
  d1 = degp[1].reshape(n_pad, 1)
  out_pad = pl.pallas_call(
      _tc_finish,
      grid=(grid,),
      in_specs=[
          pl.BlockSpec((_RBLK, d), lambda i: (i, 0)),
          pl.BlockSpec((_RBLK, d), lambda i: (i, 0)),
          pl.BlockSpec((_RBLK, 1), lambda i: (i, 0)),
          pl.BlockSpec((_RBLK, 1), lambda i: (i, 0)),
          pl.BlockSpec((d, d), lambda i: (0, 0)),
          pl.BlockSpec((1, d), lambda i: (0, 0)),
      ],
      out_specs=pl.BlockSpec((_RBLK, d), lambda i: (i, 0)),
      out_shape=jax.ShapeDtypeStruct((n_pad, d), jnp.float32),
  )(part[0], part[1], d0, d1, W, b.reshape(1, d))
  return out_pad[:n]


# speedup vs baseline: 1.3421x; 1.3421x over previous
import functools

import jax
import jax.numpy as jnp
from jax import lax
from jax.experimental import pallas as pl
from jax.experimental.pallas import tpu as pltpu
from jax.experimental.pallas import tpu_sc as plsc

_NC = 2
_NS = 16
_NW = _NC * _NS
_B = 128
_RBLK = 512


def _make_sc_segsum(n, d, n_pad, nb):
  rows_per_sub = n_pad // _NS
  mesh = plsc.VectorSubcoreMesh(core_axis_name="c", subcore_axis_name="s")

  @functools.partial(
      pl.kernel,
      out_type=(
          jax.ShapeDtypeStruct((_NC, n_pad, d), jnp.float32),
          jax.ShapeDtypeStruct((_NC, n_pad), jnp.float32),
      ),
      mesh=mesh,
      scratch_types=[
          pltpu.VMEM((nb, _B), jnp.int32),
          pltpu.VMEM((nb, _B), jnp.int32),
          pltpu.VMEM((_B, d), jnp.float32),
          pltpu.VMEM((_B,), jnp.float32),
          pltpu.VMEM_SHARED((n_pad, d), jnp.float32),
          pltpu.VMEM_SHARED((n_pad,), jnp.float32),
          pltpu.SemaphoreType.DMA,
      ],
  )
  def sc_segsum(x_h, src_h, dst_h, zr_h, zd_h, part_h, degp_h,
                src_v, dst_v, rows_v, ones_v, acc_sh, deg_sh, sem):
    c = lax.axis_index("c")
    s = lax.axis_index("s")
    w = c * _NS + s

    pltpu.sync_copy(zr_h.at[pl.ds(s * rows_per_sub, rows_per_sub)],
                    acc_sh.at[pl.ds(s * rows_per_sub, rows_per_sub)])

    @pl.when(s == 0)
    def _zero_deg():
      pltpu.sync_copy(zd_h, deg_sh)

    for k in range(_B // 16):
      ones_v[pl.ds(16 * k, 16)] = jnp.ones((16,), jnp.float32)

    pltpu.sync_copy(src_h.at[w], src_v)
    pltpu.sync_copy(dst_h.at[w], dst_v)
    plsc.subcore_barrier()

    def body(i, carry):
      pltpu.async_copy(x_h.at[src_v.at[i]], rows_v, sem).wait()
      pltpu.sync_copy(rows_v, acc_sh.at[dst_v.at[i]], add=True)
      pltpu.sync_copy(ones_v, deg_sh.at[dst_v.at[i]], add=True)
      return carry

    lax.fori_loop(0, nb, body, 0)
    plsc.subcore_barrier()

    pltpu.sync_copy(acc_sh.at[pl.ds(s * rows_per_sub, rows_per_sub)],
                    part_h.at[c, pl.ds(s * rows_per_sub, rows_per_sub)])

    @pl.when(s == 0)
    def _write_deg():
      pltpu.sync_copy(deg_sh, degp_h.at[c])

  return sc_segsum


def _tc_finish(p0_ref, p1_ref, d0_ref, d1_ref, w_ref, b_ref, o_ref):
  ssum = p0_ref[...] + p1_ref[...]
  deg = d0_ref[...] + d1_ref[...]
  inv = 1.0 / jnp.maximum(deg, 1.0)
  sn = ssum * inv
  h = jnp.dot(sn, w_ref[...], preferred_element_type=jnp.float32)
  h = h + b_ref[...]
  o_ref[...] = jnp.where(h >= 0.0, h, 0.01 * h)


def kernel(x, edge_index, W, b):
  n, d = x.shape
  e = edge_index.shape[1]
  ept = -(-e // _NW)
  nb = -(-ept // _B)
  e_pad = _NW * nb * _B
  n_pad = (n // _RBLK + 1) * _RBLK

  src = edge_index[0]
  dst = edge_index[1]
  pad = e_pad - e
  src_r = jnp.pad(src, (0, pad)).reshape(_NW, nb, _B)
  dst_r = jnp.pad(dst, (0, pad), constant_values=n).reshape(_NW, nb, _B)
  zrows = jnp.zeros((n_pad, d), jnp.float32)
  zdeg = jnp.zeros((n_pad,), jnp.float32)

  part, degp = _make_sc_segsum(n, d, n_pad, nb)(x, src_r, dst_r, zrows, zdeg)

  grid = n_pad // _RBLK
  d0 = degp[0].reshape(n_pad, 1)
  d1 = degp[1].reshape(n_pad, 1)
  out_pad = pl.pallas_call(
      _tc_finish,
      grid=(grid,),
      in_specs=[
          pl.BlockSpec((_RBLK, d), lambda i: (i, 0)),
          pl.BlockSpec((_RBLK, d), lambda i: (i, 0)),
          pl.BlockSpec((_RBLK, 1), lambda i: (i, 0)),
          pl.BlockSpec((_RBLK, 1), lambda i: (i, 0)),
          pl.BlockSpec((d, d), lambda i: (0, 0)),
          pl.BlockSpec((1, d), lambda i: (0, 0)),
      ],
      out_specs=pl.BlockSpec((_RBLK, d), lambda i: (i, 0)),
      out_shape=jax.ShapeDtypeStruct((n_pad, d), jnp.float32),
  )(part[0], part[1], d0, d1, W, b.reshape(1, d))
  return out_pad[:n]
